# baseline (speedup 1.0000x reference)
import jax
import jax.numpy as jnp
from jax.experimental import pallas as pl
from jax.experimental.pallas import tpu as pltpu

B = 1
S = 2048
HID = 2048
NH = 16
NKV = 8
HD = 128
WINDOW = 1024
SOFTCAP = 50.0
SCALING = HD ** -0.5
LOG2E = 1.4426950408889634

BN_P = 512
N_QKV = (NH + 2 * NKV) * HD
N_ROT_BLOCKS = (NH + NKV) * HD // BN_P
Q_BLOCKS = NH * HD // BN_P


def _qkv_kernel(x_ref, w_ref, cos_ref, sin_ref, o_ref, xbf_ref):
    j = pl.program_id(0)

    @pl.when(j == 0)
    def _cast_x():
        xbf_ref[...] = x_ref[...].astype(jnp.bfloat16)

    y = jax.lax.dot_general(
        xbf_ref[...], w_ref[...].astype(jnp.bfloat16),
        (((1,), (1,)), ((), ())),
        preferred_element_type=jnp.float32)

    @pl.when(j < N_ROT_BLOCKS)
    def _rot():
        scale = jnp.where(j < Q_BLOCKS, SCALING / SOFTCAP, 1.0)
        yh = y.reshape(S, BN_P // HD, HD)
        rot_half = jnp.concatenate(
            [-yh[..., HD // 2:], yh[..., :HD // 2]], axis=-1)
        cos = cos_ref[...][:, None, :] * scale
        sin = sin_ref[...][:, None, :] * scale
        o_ref[...] = (yh * cos + rot_half * sin).reshape(
            S, BN_P).astype(jnp.bfloat16)

    @pl.when(j >= N_ROT_BLOCKS)
    def _plain():
        o_ref[...] = y.astype(jnp.bfloat16)


def _qkv_proj(x2d, w_qkv, cos_full, sin_full):
    return pl.pallas_call(
        _qkv_kernel,
        grid=(N_QKV // BN_P,),
        in_specs=[
            pl.BlockSpec((S, HID), lambda j: (0, 0)),
            pl.BlockSpec((BN_P, HID), lambda j: (j, 0)),
            pl.BlockSpec((S, HD), lambda j: (0, 0)),
            pl.BlockSpec((S, HD), lambda j: (0, 0)),
        ],
        out_specs=pl.BlockSpec((S, BN_P), lambda j: (0, j)),
        out_shape=jax.ShapeDtypeStruct((S, N_QKV), jnp.bfloat16),
        scratch_shapes=[pltpu.VMEM((S, HID), jnp.bfloat16)],
    )(x2d, w_qkv, cos_full, sin_full)


BQ = 512
BK = 512
NT = WINDOW // BK + 1
C2 = SOFTCAP * LOG2E
NREP = NH // NKV


def _attn_kernel(q_ref, k_ref, v_ref, o_ref, denom_ref, acc_ref, mask_ref):
    g = pl.program_id(0)
    qb = pl.program_id(1)
    t = pl.program_id(2)

    @pl.when((g == 0) & (qb == 0) & (t == 0))
    def _build_masks():
        ri = jax.lax.broadcasted_iota(jnp.int32, (BQ, BK), 0)
        ci = jax.lax.broadcasted_iota(jnp.int32, (BQ, BK), 1)
        mask_ref[0] = (ci > ri).astype(jnp.float32)
        for mid in range(1, NT - 1):
            mask_ref[mid] = jnp.ones((BQ, BK), jnp.float32)
        mask_ref[NT - 1] = (ci <= ri).astype(jnp.float32)

    @pl.when(t == 0)
    def _init():
        denom_ref[...] = jnp.zeros_like(denom_ref)
        acc_ref[...] = jnp.zeros_like(acc_ref)

    @pl.when(qb + t - (NT - 1) >= 0)
    def _compute():
        k = k_ref[...]
        v = v_ref[...]
        m = mask_ref[t]
        for s in range(NREP):
            u = jax.lax.dot_general(
                q_ref[:, s * HD:(s + 1) * HD], k, (((1,), (1,)), ((), ())),
                preferred_element_type=jnp.float32)
            p = jnp.exp2(jnp.tanh(u) * C2) * m
            denom_ref[:, s:s + 1] += jnp.sum(p, axis=1, keepdims=True)
            acc_ref[:, s * HD:(s + 1) * HD] += jax.lax.dot_general(
                p.astype(jnp.bfloat16), v, (((1,), (0,)), ((), ())),
                preferred_element_type=jnp.float32)

    @pl.when(t == NT - 1)
    def _finish():
        for s in range(NREP):
            o_ref[:, s * HD:(s + 1) * HD] = (
                acc_ref[:, s * HD:(s + 1) * HD] / denom_ref[:, s:s + 1]
            ).astype(jnp.bfloat16)


def _attention(qkv):
    return pl.pallas_call(
        _attn_kernel,
        grid=(NKV, S // BQ, NT),
        in_specs=[
            pl.BlockSpec((BQ, NREP * HD), lambda g, qb, t: (qb, g)),
            pl.BlockSpec((BK, HD), lambda g, qb, t:
                         (jnp.maximum(qb + t - (NT - 1), 0), NH + g)),
            pl.BlockSpec((BK, HD), lambda g, qb, t:
                         (jnp.maximum(qb + t - (NT - 1), 0), NH + NKV + g)),
        ],
        out_specs=pl.BlockSpec((BQ, NREP * HD), lambda g, qb, t: (qb, g)),
        out_shape=jax.ShapeDtypeStruct((S, NH * HD), jnp.bfloat16),
        scratch_shapes=[pltpu.VMEM((BQ, NREP), jnp.float32),
                        pltpu.VMEM((BQ, NREP * HD), jnp.float32),
                        pltpu.VMEM((NT, BQ, BK), jnp.float32)],
    )(qkv, qkv, qkv)


BM_O = 512


def _oproj_kernel(a_ref, w_ref, o_ref, wbf_ref):
    @pl.when(pl.program_id(0) == 0)
    def _cast_w():
        wbf_ref[...] = w_ref[...].astype(jnp.bfloat16)

    o_ref[...] = jax.lax.dot_general(
        a_ref[...], wbf_ref[...],
        (((1,), (1,)), ((), ())),
        preferred_element_type=jnp.float32)


def _out_proj(attn, w_o):
    return pl.pallas_call(
        _oproj_kernel,
        grid=(S // BM_O,),
        in_specs=[
            pl.BlockSpec((BM_O, NH * HD), lambda i: (i, 0)),
            pl.BlockSpec((HID, NH * HD), lambda i: (0, 0)),
        ],
        out_specs=pl.BlockSpec((BM_O, HID), lambda i: (i, 0)),
        out_shape=jax.ShapeDtypeStruct((S, HID), jnp.float32),
        scratch_shapes=[pltpu.VMEM((HID, NH * HD), jnp.bfloat16)],
    )(attn, w_o)


@jax.jit
def kernel(x, freqs_cis, w_qkv, w_o):
    x2d = x.reshape(S, HID)
    cos = freqs_cis[..., 0]
    sin = freqs_cis[..., 1]
    cos_full = jnp.concatenate([cos, cos], axis=-1)
    sin_full = jnp.concatenate([sin, sin], axis=-1)
    qkv = _qkv_proj(x2d, w_qkv, cos_full, sin_full)
    attn = _attention(qkv)
    out = _out_proj(attn, w_o)
    return out.reshape(B, S, HID)
